# Initial kernel scaffold; baseline (speedup 1.0000x reference)
import jax
import jax.numpy as jnp
from jax.experimental import pallas as pl


def kernel(node_attr, edge_index, edge_attr, edge_sh, fc_w1, fc_b1, fc_w2, fc_b2):
    raise NotImplementedError("write your pallas kernel here")


# speedup vs baseline: 7.1090x; 7.1090x over previous
import functools

import jax
import jax.numpy as jnp
import numpy as np
from jax import lax
from jax.experimental import pallas as pl
from jax.experimental.pallas import tpu as pltpu
from jax.experimental.pallas import tpu_sc as plsc

N = 10000
E = 160000
DIN = 42
DP = 48
DE = 16
HID = 16
WNUM = 468
CW = 16

NC, NS = 2, 16
NW = NC * NS
CH = 128
NCHT = E // CH
NPAD = 10240
RPS = NPAD // NS

BE = 6400
BS = BE // 128
GRID = E // BE

_INV3 = float(1.0 / np.sqrt(3.0))
_INV2 = float(1.0 / np.sqrt(2.0))


def _gather_body(table_hbm, idx_hbm, out_hbm, idx_v, rows_v, sem):
  c = lax.axis_index("c")
  s = lax.axis_index("s")
  wid = s * NC + c
  nch = 39 + jnp.where(wid < NCHT - 39 * NW, 1, 0)

  def step(t, carry):
    g = t * NW + wid
    pltpu.sync_copy(idx_hbm.at[g], idx_v)
    pltpu.async_copy(table_hbm.at[idx_v.at[0]], rows_v, sem).wait()
    pltpu.sync_copy(rows_v, out_hbm.at[pl.ds(g * CH, CH)])
    return carry

  lax.fori_loop(0, nch, step, 0)


def _sc_gather(table, dst3):
  mesh = plsc.VectorSubcoreMesh(core_axis_name="c", subcore_axis_name="s")
  return pl.kernel(
      _gather_body,
      out_type=jax.ShapeDtypeStruct((E, DP), jnp.float32),
      mesh=mesh,
      compiler_params=pltpu.CompilerParams(use_tc_tiling_on_sc=False),
      scratch_types=[
          pltpu.VMEM((1, CH), jnp.int32),
          pltpu.VMEM((CH, DP), jnp.float32),
          pltpu.SemaphoreType.DMA,
      ],
  )(table, dst3)


def _scatter_body(tp_hbm, src_hbm, zsum_hbm, zcnt_hbm, ones_hbm,
                  psum_hbm, pcnt_hbm,
                  idx_v, rows_v, ones_v, zb1_v, zb2_v,
                  ssum, scnt):
  c = lax.axis_index("c")
  s = lax.axis_index("s")
  wid = s * NC + c
  nch = 39 + jnp.where(wid < NCHT - 39 * NW, 1, 0)

  pltpu.sync_copy(zsum_hbm, zb1_v)
  pltpu.sync_copy(zcnt_hbm, zb2_v)
  pltpu.sync_copy(zb1_v, ssum.at[pl.ds(s * RPS, RPS)])
  pltpu.sync_copy(zb2_v, scnt.at[pl.ds(s * RPS, RPS)])
  pltpu.sync_copy(ones_hbm, ones_v)
  plsc.subcore_barrier()

  def step(t, carry):
    g = t * NW + wid
    pltpu.sync_copy(src_hbm.at[g], idx_v)
    pltpu.sync_copy(tp_hbm.at[pl.ds(g * CH, CH)], rows_v)
    pltpu.sync_copy(rows_v, ssum.at[idx_v.at[0]], add=True)
    pltpu.sync_copy(ones_v, scnt.at[idx_v.at[0]], add=True)
    return carry

  lax.fori_loop(0, nch, step, 0)
  plsc.subcore_barrier()

  pltpu.sync_copy(ssum.at[pl.ds(s * RPS, RPS)], zb1_v)
  pltpu.sync_copy(scnt.at[pl.ds(s * RPS, RPS)], zb2_v)
  pltpu.sync_copy(zb1_v, psum_hbm.at[c, pl.ds(s * RPS, RPS)])
  pltpu.sync_copy(zb2_v, pcnt_hbm.at[c, pl.ds(s * RPS, RPS)])


def _sc_scatter(tp, src3, zsum, zcnt, ones_blk):
  mesh = plsc.VectorSubcoreMesh(core_axis_name="c", subcore_axis_name="s")
  return pl.kernel(
      _scatter_body,
      out_type=(
          jax.ShapeDtypeStruct((NC, NPAD, DP), jnp.float32),
          jax.ShapeDtypeStruct((NC, NPAD, CW), jnp.float32),
      ),
      mesh=mesh,
      compiler_params=pltpu.CompilerParams(use_tc_tiling_on_sc=False),
      scratch_types=[
          pltpu.VMEM((1, CH), jnp.int32),
          pltpu.VMEM((CH, DP), jnp.float32),
          pltpu.VMEM((CH, CW), jnp.float32),
          pltpu.VMEM((RPS, DP), jnp.float32),
          pltpu.VMEM((RPS, CW), jnp.float32),
          pltpu.VMEM_SHARED((NPAD, DP), jnp.float32),
          pltpu.VMEM_SHARED((NPAD, CW), jnp.float32),
      ],
  )(tp, src3, zsum, zcnt, ones_blk)


def _dense_body(eaT_ref, shT_ref, x_ref, w1t_ref, b1_ref, w2t_ref, b2_ref,
                eye_ref, tp_ref):
  f32 = jnp.float32
  ea = eaT_ref[...]
  h = jnp.maximum(
      lax.dot_general(w1t_ref[...], ea, (((1,), (0,)), ((), ())),
                      preferred_element_type=f32) + b1_ref[...], 0.0)
  w2d = lax.dot_general(w2t_ref[...], h, (((1,), (0,)), ((), ())),
                        preferred_element_type=f32) + b2_ref[...]
  eye = eye_ref[...]
  x = x_ref[...]
  xt = lax.dot_general(eye, x, (((1,), (1,)), ((), ())),
                       preferred_element_type=f32)

  X = xt.reshape(DP, BS, 128)
  S = shT_ref[...].reshape(4, BS, 128)
  W = w2d.reshape(WNUM, BS, 128)
  SH0 = S[0]
  SH1 = [S[1], S[2], S[3]]

  def XP(j):
    return X[j]

  def WP(r):
    return W[r]

  f0e = [XP(i) * SH0 for i in range(16)]
  f0e += [(XP(16 + 3 * i) * SH1[0] + XP(17 + 3 * i) * SH1[1]
           + XP(18 + 3 * i) * SH1[2]) * _INV3 for i in range(4)]

  o1o = [[XP(i) * SH1[cc] for cc in range(3)] for i in range(16)]
  o1o += [[XP(16 + 3 * i + cc) * SH0 for cc in range(3)] for i in range(4)]
  for i in range(4):
    a = [XP(28 + 3 * i + cc) for cc in range(3)]
    o1o.append([(a[(cc + 1) % 3] * SH1[(cc + 2) % 3]
                 - a[(cc + 2) % 3] * SH1[(cc + 1) % 3]) * _INV2
                for cc in range(3)])

  o1e = []
  for i in range(4):
    a = [XP(16 + 3 * i + cc) for cc in range(3)]
    o1e.append([(a[(cc + 1) % 3] * SH1[(cc + 2) % 3]
                 - a[(cc + 2) % 3] * SH1[(cc + 1) % 3]) * _INV2
                for cc in range(3)])
  o1e += [[XP(28 + 3 * i + cc) * SH0 for cc in range(3)] for i in range(4)]
  o1e += [[XP(40 + i) * SH1[cc] for cc in range(3)] for i in range(2)]

  f0o = [(XP(28 + 3 * i) * SH1[0] + XP(29 + 3 * i) * SH1[1]
          + XP(30 + 3 * i) * SH1[2]) * _INV3 for i in range(4)]
  f0o += [XP(40 + i) * SH0 for i in range(2)]

  planes = []
  for o in range(16):
    acc = f0e[0] * WP(o)
    for i in range(1, 20):
      acc += f0e[i] * WP(i * 16 + o)
    planes.append(acc)
  for o in range(4):
    for cc in range(3):
      acc = o1o[0][cc] * WP(320 + o)
      for i in range(1, 24):
        acc += o1o[i][cc] * WP(320 + i * 4 + o)
      planes.append(acc)
  for o in range(4):
    for cc in range(3):
      acc = o1e[0][cc] * WP(416 + o)
      for i in range(1, 10):
        acc += o1e[i][cc] * WP(416 + i * 4 + o)
      planes.append(acc)
  for o in range(2):
    acc = f0o[0] * WP(456 + o)
    for i in range(1, 6):
      acc += f0o[i] * WP(456 + i * 2 + o)
    planes.append(acc)

  zero = jnp.zeros_like(planes[0])
  planes += [zero] * (DP - DIN)
  tpt = jnp.stack(planes, axis=0).reshape(DP, BE)
  tp_ref[...] = lax.dot_general(tpt, eye, (((0,), (0,)), ((), ())),
                                preferred_element_type=f32)


def _tc_dense(eaT, shT, x, w1t, b1c, w2t, b2c, eye48):
  return pl.pallas_call(
      _dense_body,
      grid=(GRID,),
      in_specs=[
          pl.BlockSpec((DE, BE), lambda i: (0, i)),
          pl.BlockSpec((4, BE), lambda i: (0, i)),
          pl.BlockSpec((BE, DP), lambda i: (i, 0)),
          pl.BlockSpec((DE, DE), lambda i: (0, 0)),
          pl.BlockSpec((DE, 1), lambda i: (0, 0)),
          pl.BlockSpec((WNUM, DE), lambda i: (0, 0)),
          pl.BlockSpec((WNUM, 1), lambda i: (0, 0)),
          pl.BlockSpec((DP, DP), lambda i: (0, 0)),
      ],
      out_specs=pl.BlockSpec((BE, DP), lambda i: (i, 0)),
      out_shape=jax.ShapeDtypeStruct((E, DP), jnp.float32),
  )(eaT, shT, x, w1t, b1c, w2t, b2c, eye48)


def _combine_body(ps_ref, pc_ref, na_ref, out_ref):
  sums = ps_ref[0] + ps_ref[1]
  cnt = pc_ref[0, :, 0:1] + pc_ref[1, :, 0:1]
  out_ref[...] = sums[:, :DIN] / jnp.maximum(cnt, 1.0) + na_ref[...]


def _tc_combine(psum, pcnt, na_pad):
  bn = 1024
  return pl.pallas_call(
      _combine_body,
      grid=(NPAD // bn,),
      in_specs=[
          pl.BlockSpec((NC, bn, DP), lambda i: (0, i, 0)),
          pl.BlockSpec((NC, bn, CW), lambda i: (0, i, 0)),
          pl.BlockSpec((bn, DIN), lambda i: (i, 0)),
      ],
      out_specs=pl.BlockSpec((bn, DIN), lambda i: (i, 0)),
      out_shape=jax.ShapeDtypeStruct((NPAD, DIN), jnp.float32),
  )(psum, pcnt, na_pad)


@jax.jit
def kernel(node_attr, edge_index, edge_attr, edge_sh,
           fc_w1, fc_b1, fc_w2, fc_b2):
  f32 = jnp.float32
  node_attr = node_attr.astype(f32)
  edge_src = edge_index[0].astype(jnp.int32)
  edge_dst = edge_index[1].astype(jnp.int32)

  table = jnp.pad(node_attr, ((0, 0), (0, DP - DIN)))
  dst3 = edge_dst.reshape(NCHT, 1, CH)
  src3 = edge_src.reshape(NCHT, 1, CH)

  scale = np.concatenate([
      np.full(320, 1.0 / np.sqrt(20.0)),
      np.full(96, 1.0 / np.sqrt(24.0)),
      np.full(40, 1.0 / np.sqrt(10.0)),
      np.full(12, 1.0 / np.sqrt(6.0)),
  ]).astype(np.float32)
  w1t = fc_w1.astype(f32).T
  b1c = fc_b1.astype(f32)[:, None]
  w2t = (fc_w2.astype(f32) * scale[None, :]).T
  b2c = (fc_b2.astype(f32) * scale)[:, None]
  eye48 = jnp.eye(DP, dtype=f32)

  x = _sc_gather(table, dst3)
  eaT = edge_attr.astype(f32).T
  shT = edge_sh.astype(f32).T
  tp = _tc_dense(eaT, shT, x, w1t, b1c, w2t, b2c, eye48)

  zsum = jnp.zeros((RPS, DP), f32)
  zcnt = jnp.zeros((RPS, CW), f32)
  ones_blk = jnp.ones((CH, CW), f32)
  psum, pcnt = _sc_scatter(tp, src3, zsum, zcnt, ones_blk)

  na_pad = jnp.pad(node_attr, ((0, NPAD - N), (0, 0)))
  out = _tc_combine(psum, pcnt, na_pad)
  return out[:N]
